# Initial kernel scaffold; baseline (speedup 1.0000x reference)
import jax
import jax.numpy as jnp
from jax.experimental import pallas as pl


def kernel(x, edge_index, W1, b1, W2, b2, Wlin, blin):
    raise NotImplementedError("write your pallas kernel here")


# speedup vs baseline: 12.0063x; 12.0063x over previous
import functools

import jax
import jax.numpy as jnp
from jax import lax
from jax.experimental import pallas as pl
from jax.experimental.pallas import tpu as pltpu
from jax.experimental.pallas import tpu_sc as plsc

N_NODES = 10000
D_FEAT = 128
HIDDEN = 128
N_EDGES = 320000
N_PAD = 10240

NC = 2
NS = 16
NW = NC * NS
EDGES_PER_W = N_EDGES // NW
CHUNK = 80
N_CHUNKS = EDGES_PER_W // CHUNK
ROWS_PER_TILE = N_PAD // NS

_mesh = plsc.VectorSubcoreMesh(core_axis_name="c", subcore_axis_name="s")


@functools.partial(
    pl.kernel,
    out_type=jax.ShapeDtypeStruct((NC * N_PAD, HIDDEN), jnp.float32),
    mesh=_mesh,
    scratch_types=[
        pltpu.VMEM((CHUNK,), jnp.int32),
        pltpu.VMEM((CHUNK, HIDDEN), jnp.float32),
        pltpu.VMEM_SHARED((N_PAD, HIDDEN), jnp.float32),
        pltpu.SemaphoreType.DMA,
    ],
)
def _deg(ones_hbm, dst_hbm, out_hbm, idx_d, rows_v, acc, sem):
    cid = lax.axis_index("c")
    sid = lax.axis_index("s")
    wid = sid * NC + cid
    r0 = sid * ROWS_PER_TILE
    o0 = cid * N_PAD

    pltpu.sync_copy(ones_hbm, rows_v)

    def init(k, _):
        pltpu.sync_copy(rows_v, acc.at[pl.ds(r0 + k * CHUNK, CHUNK)])
        return 0

    lax.fori_loop(0, ROWS_PER_TILE // CHUNK, init, 0)
    plsc.subcore_barrier()

    def body(j, _):
        base = wid * EDGES_PER_W + j * CHUNK
        pltpu.sync_copy(dst_hbm.at[pl.ds(base, CHUNK)], idx_d)
        pltpu.sync_copy(rows_v, acc.at[idx_d], add=True)
        return 0

    lax.fori_loop(0, N_CHUNKS, body, 0)
    plsc.subcore_barrier()

    def writeback(k, _):
        b = r0 + k * CHUNK
        pltpu.sync_copy(acc.at[pl.ds(b, CHUNK)], rows_v)
        pltpu.sync_copy(rows_v, out_hbm.at[pl.ds(o0 + b, CHUNK)])
        return 0

    lax.fori_loop(0, ROWS_PER_TILE // CHUNK, writeback, 0)


@functools.partial(
    pl.kernel,
    out_type=jax.ShapeDtypeStruct((NC * N_PAD, HIDDEN), jnp.float32),
    mesh=_mesh,
    scratch_types=[
        pltpu.VMEM((CHUNK,), jnp.int32),
        pltpu.VMEM((CHUNK,), jnp.int32),
        pltpu.VMEM((CHUNK, HIDDEN), jnp.float32),
        pltpu.VMEM_SHARED((N_PAD, HIDDEN), jnp.float32),
        pltpu.SemaphoreType.DMA,
    ],
)
def _agg(hs_hbm, src_hbm, dst_hbm, out_hbm, idx_s, idx_d, rows_v, acc, sem):
    cid = lax.axis_index("c")
    sid = lax.axis_index("s")
    wid = sid * NC + cid
    r0 = sid * ROWS_PER_TILE
    o0 = cid * N_PAD

    def init(k, _):
        b = r0 + k * CHUNK
        pltpu.sync_copy(hs_hbm.at[pl.ds(b, CHUNK)], rows_v)
        pltpu.sync_copy(rows_v, acc.at[pl.ds(b, CHUNK)])
        return 0

    lax.fori_loop(0, ROWS_PER_TILE // CHUNK, init, 0)
    plsc.subcore_barrier()

    def body(j, _):
        base = wid * EDGES_PER_W + j * CHUNK
        pltpu.sync_copy(src_hbm.at[pl.ds(base, CHUNK)], idx_s)
        pltpu.sync_copy(dst_hbm.at[pl.ds(base, CHUNK)], idx_d)
        pltpu.async_copy(hs_hbm.at[idx_s], rows_v, sem).wait()
        pltpu.sync_copy(rows_v, acc.at[idx_d], add=True)
        return 0

    lax.fori_loop(0, N_CHUNKS, body, 0)
    plsc.subcore_barrier()

    def writeback(k, _):
        b = r0 + k * CHUNK
        pltpu.sync_copy(acc.at[pl.ds(b, CHUNK)], rows_v)
        pltpu.sync_copy(rows_v, out_hbm.at[pl.ds(o0 + b, CHUNK)])
        return 0

    lax.fori_loop(0, ROWS_PER_TILE // CHUNK, writeback, 0)


ROWS_BLK = 1024
_GRID = N_PAD // ROWS_BLK
_NBLK = N_PAD // ROWS_BLK


def _scale1_body(pd0_ref, pd1_ref, x_ref, w_ref, o_ref):
    dinv = lax.rsqrt(pd0_ref[...] + pd1_ref[...] - 1.0)
    h = jnp.dot(x_ref[...], w_ref[...], preferred_element_type=jnp.float32)
    o_ref[...] = h * dinv


def _mid_body(pd0_ref, pd1_ref, p0_ref, p1_ref, hs_ref, b_ref, w_ref, o_ref):
    dinv = lax.rsqrt(pd0_ref[...] + pd1_ref[...] - 1.0)
    agg = p0_ref[...] + p1_ref[...] - hs_ref[...]
    h = jnp.maximum(agg * dinv + b_ref[...], 0.0)
    o_ref[...] = jnp.dot(h, w_ref[...], preferred_element_type=jnp.float32) * dinv


def _head_body(pd0_ref, pd1_ref, p0_ref, p1_ref, hs_ref, b_ref, w_ref,
               blin_ref, o_ref):
    dinv = lax.rsqrt(pd0_ref[...] + pd1_ref[...] - 1.0)
    agg = p0_ref[...] + p1_ref[...] - hs_ref[...]
    h = jnp.maximum(agg * dinv + b_ref[...], 0.0)
    o_ref[...] = (jnp.dot(h, w_ref[...], preferred_element_type=jnp.float32)
                  + blin_ref[...])


def _half0_spec():
    return pl.BlockSpec((ROWS_BLK, HIDDEN), lambda i: (i, 0))


def _half1_spec():
    return pl.BlockSpec((ROWS_BLK, HIDDEN), lambda i: (i + _NBLK, 0))


def _rows_spec(width):
    return pl.BlockSpec((ROWS_BLK, width), lambda i: (i, 0))


def _full_spec(shape):
    return pl.BlockSpec(shape, lambda i: tuple(0 for _ in shape))


def kernel(x, edge_index, W1, b1, W2, b2, Wlin, blin):
    src = edge_index[0].astype(jnp.int32)
    dst = edge_index[1].astype(jnp.int32)
    xp = jnp.pad(x, ((0, N_PAD - N_NODES), (0, 0)))
    ones = jnp.ones((CHUNK, HIDDEN), jnp.float32)

    pdeg = _deg(ones, dst)

    hs1 = pl.pallas_call(
        _scale1_body,
        grid=(_GRID,),
        in_specs=[_half0_spec(), _half1_spec(), _rows_spec(D_FEAT),
                  _full_spec((D_FEAT, HIDDEN))],
        out_specs=_rows_spec(HIDDEN),
        out_shape=jax.ShapeDtypeStruct((N_PAD, HIDDEN), jnp.float32),
    )(pdeg, pdeg, xp, W1)

    p1 = _agg(hs1, src, dst)

    hs2 = pl.pallas_call(
        _mid_body,
        grid=(_GRID,),
        in_specs=[_half0_spec(), _half1_spec(), _half0_spec(), _half1_spec(),
                  _rows_spec(HIDDEN), _full_spec((1, HIDDEN)),
                  _full_spec((HIDDEN, HIDDEN))],
        out_specs=_rows_spec(HIDDEN),
        out_shape=jax.ShapeDtypeStruct((N_PAD, HIDDEN), jnp.float32),
    )(pdeg, pdeg, p1, p1, hs1, b1.reshape(1, HIDDEN), W2)

    p2 = _agg(hs2, src, dst)

    out = pl.pallas_call(
        _head_body,
        grid=(_GRID,),
        in_specs=[_half0_spec(), _half1_spec(), _half0_spec(), _half1_spec(),
                  _rows_spec(HIDDEN), _full_spec((1, HIDDEN)),
                  _full_spec((HIDDEN, 1)), _full_spec((1, 1))],
        out_specs=_rows_spec(1),
        out_shape=jax.ShapeDtypeStruct((N_PAD, 1), jnp.float32),
    )(pdeg, pdeg, p2, p2, hs2, b2.reshape(1, HIDDEN), Wlin, blin.reshape(1, 1))

    return out[:N_NODES, 0]


# speedup vs baseline: 19.7067x; 1.6414x over previous
import functools

import jax
import jax.numpy as jnp
from jax import lax
from jax.experimental import pallas as pl
from jax.experimental.pallas import tpu as pltpu
from jax.experimental.pallas import tpu_sc as plsc

N_NODES = 10000
D_FEAT = 128
HIDDEN = 128
N_EDGES = 320000
N_PAD = 10240

NC = 2
NS = 16
NW = NC * NS
EDGES_PER_W = N_EDGES // NW
CHUNK = 80
N_CHUNKS = EDGES_PER_W // CHUNK
ROWS_PER_TILE = N_PAD // NS

_mesh = plsc.VectorSubcoreMesh(core_axis_name="c", subcore_axis_name="s")


@functools.partial(
    pl.kernel,
    out_type=jax.ShapeDtypeStruct((NC * N_PAD, HIDDEN), jnp.float32),
    mesh=_mesh,
    scratch_types=[
        pltpu.VMEM((CHUNK,), jnp.int32),
        pltpu.VMEM((CHUNK, HIDDEN), jnp.float32),
        pltpu.VMEM_SHARED((N_PAD, HIDDEN), jnp.float32),
        pltpu.SemaphoreType.DMA,
    ],
)
def _deg(ones_hbm, dst_hbm, out_hbm, idx_d, rows_v, acc, sem):
    cid = lax.axis_index("c")
    sid = lax.axis_index("s")
    wid = sid * NC + cid
    r0 = sid * ROWS_PER_TILE
    o0 = cid * N_PAD

    pltpu.sync_copy(ones_hbm, rows_v)

    def init(k, _):
        pltpu.sync_copy(rows_v, acc.at[pl.ds(r0 + k * CHUNK, CHUNK)])
        return 0

    lax.fori_loop(0, ROWS_PER_TILE // CHUNK, init, 0)
    plsc.subcore_barrier()

    def body(j, _):
        base = wid * EDGES_PER_W + j * CHUNK
        pltpu.sync_copy(dst_hbm.at[pl.ds(base, CHUNK)], idx_d)
        pltpu.sync_copy(rows_v, acc.at[idx_d], add=True)
        return 0

    lax.fori_loop(0, N_CHUNKS, body, 0)
    plsc.subcore_barrier()

    def writeback(k, _):
        b = r0 + k * CHUNK
        pltpu.sync_copy(acc.at[pl.ds(b, CHUNK)], rows_v)
        pltpu.sync_copy(rows_v, out_hbm.at[pl.ds(o0 + b, CHUNK)])
        return 0

    lax.fori_loop(0, ROWS_PER_TILE // CHUNK, writeback, 0)


N_PAIRS = (N_CHUNKS - 1) // 2


@functools.partial(
    pl.kernel,
    out_type=jax.ShapeDtypeStruct((NC * N_PAD, HIDDEN), jnp.float32),
    mesh=_mesh,
    scratch_types=[
        pltpu.VMEM((CHUNK,), jnp.int32),
        pltpu.VMEM((CHUNK,), jnp.int32),
        pltpu.VMEM((CHUNK,), jnp.int32),
        pltpu.VMEM((CHUNK,), jnp.int32),
        pltpu.VMEM((CHUNK, HIDDEN), jnp.float32),
        pltpu.VMEM((CHUNK, HIDDEN), jnp.float32),
        pltpu.VMEM_SHARED((N_PAD, HIDDEN), jnp.float32),
        pltpu.SemaphoreType.DMA,
        pltpu.SemaphoreType.DMA,
    ],
)
def _agg(hs_hbm, src_hbm, dst_hbm, out_hbm, sA, sB, dA, dB, bufA, bufB, acc,
         sem_g, sem_s):
    cid = lax.axis_index("c")
    sid = lax.axis_index("s")
    wid = sid * NC + cid
    r0 = sid * ROWS_PER_TILE
    o0 = cid * N_PAD
    e0 = wid * EDGES_PER_W

    def init(k, _):
        b = r0 + k * CHUNK
        pltpu.sync_copy(hs_hbm.at[pl.ds(b, CHUNK)], bufA)
        pltpu.sync_copy(bufA, acc.at[pl.ds(b, CHUNK)])
        return 0

    lax.fori_loop(0, ROWS_PER_TILE // CHUNK, init, 0)

    pltpu.sync_copy(src_hbm.at[pl.ds(e0, CHUNK)], sA)
    pltpu.async_copy(hs_hbm.at[sA], bufA, sem_g)
    plsc.subcore_barrier()

    def _wait(sem):
        pltpu.make_async_copy(hs_hbm.at[pl.ds(0, CHUNK)], bufA, sem).wait()

    def body(k, _):
        j0 = 2 * k
        pltpu.sync_copy(dst_hbm.at[pl.ds(e0 + j0 * CHUNK, CHUNK)], dA)

        @pl.when(k > 0)
        def _wait_prev_scatter():
            _wait(sem_s)

        pltpu.sync_copy(src_hbm.at[pl.ds(e0 + (j0 + 1) * CHUNK, CHUNK)], sB)
        pltpu.async_copy(hs_hbm.at[sB], bufB, sem_g)
        _wait(sem_g)
        pltpu.async_copy(bufA, acc.at[dA], sem_s, add=True)
        pltpu.sync_copy(dst_hbm.at[pl.ds(e0 + (j0 + 1) * CHUNK, CHUNK)], dB)
        pltpu.sync_copy(src_hbm.at[pl.ds(e0 + (j0 + 2) * CHUNK, CHUNK)], sA)
        _wait(sem_s)
        pltpu.async_copy(hs_hbm.at[sA], bufA, sem_g)
        _wait(sem_g)
        pltpu.async_copy(bufB, acc.at[dB], sem_s, add=True)
        return 0

    lax.fori_loop(0, N_PAIRS, body, 0)

    pltpu.sync_copy(dst_hbm.at[pl.ds(e0 + (N_CHUNKS - 1) * CHUNK, CHUNK)], dA)
    _wait(sem_s)
    _wait(sem_g)
    pltpu.async_copy(bufA, acc.at[dA], sem_s, add=True)
    _wait(sem_s)
    plsc.subcore_barrier()

    def writeback(k, _):
        b = r0 + k * CHUNK
        pltpu.sync_copy(acc.at[pl.ds(b, CHUNK)], bufA)
        pltpu.sync_copy(bufA, out_hbm.at[pl.ds(o0 + b, CHUNK)])
        return 0

    lax.fori_loop(0, ROWS_PER_TILE // CHUNK, writeback, 0)


ROWS_BLK = 1024
_GRID = N_PAD // ROWS_BLK
_NBLK = N_PAD // ROWS_BLK


def _scale1_body(pd0_ref, pd1_ref, x_ref, w_ref, o_ref):
    dinv = lax.rsqrt(pd0_ref[...] + pd1_ref[...] - 1.0)
    h = jnp.dot(x_ref[...], w_ref[...], preferred_element_type=jnp.float32)
    o_ref[...] = h * dinv


def _mid_body(pd0_ref, pd1_ref, p0_ref, p1_ref, hs_ref, b_ref, w_ref, o_ref):
    dinv = lax.rsqrt(pd0_ref[...] + pd1_ref[...] - 1.0)
    agg = p0_ref[...] + p1_ref[...] - hs_ref[...]
    h = jnp.maximum(agg * dinv + b_ref[...], 0.0)
    o_ref[...] = jnp.dot(h, w_ref[...], preferred_element_type=jnp.float32) * dinv


def _head_body(pd0_ref, pd1_ref, p0_ref, p1_ref, hs_ref, b_ref, w_ref,
               blin_ref, o_ref):
    dinv = lax.rsqrt(pd0_ref[...] + pd1_ref[...] - 1.0)
    agg = p0_ref[...] + p1_ref[...] - hs_ref[...]
    h = jnp.maximum(agg * dinv + b_ref[...], 0.0)
    o_ref[...] = (jnp.dot(h, w_ref[...], preferred_element_type=jnp.float32)
                  + blin_ref[...])


def _half0_spec():
    return pl.BlockSpec((ROWS_BLK, HIDDEN), lambda i: (i, 0))


def _half1_spec():
    return pl.BlockSpec((ROWS_BLK, HIDDEN), lambda i: (i + _NBLK, 0))


def _rows_spec(width):
    return pl.BlockSpec((ROWS_BLK, width), lambda i: (i, 0))


def _full_spec(shape):
    return pl.BlockSpec(shape, lambda i: tuple(0 for _ in shape))


def kernel(x, edge_index, W1, b1, W2, b2, Wlin, blin):
    src = edge_index[0].astype(jnp.int32)
    dst = edge_index[1].astype(jnp.int32)
    xp = jnp.pad(x, ((0, N_PAD - N_NODES), (0, 0)))
    ones = jnp.ones((CHUNK, HIDDEN), jnp.float32)

    pdeg = _deg(ones, dst)

    hs1 = pl.pallas_call(
        _scale1_body,
        grid=(_GRID,),
        in_specs=[_half0_spec(), _half1_spec(), _rows_spec(D_FEAT),
                  _full_spec((D_FEAT, HIDDEN))],
        out_specs=_rows_spec(HIDDEN),
        out_shape=jax.ShapeDtypeStruct((N_PAD, HIDDEN), jnp.float32),
    )(pdeg, pdeg, xp, W1)

    p1 = _agg(hs1, src, dst)

    hs2 = pl.pallas_call(
        _mid_body,
        grid=(_GRID,),
        in_specs=[_half0_spec(), _half1_spec(), _half0_spec(), _half1_spec(),
                  _rows_spec(HIDDEN), _full_spec((1, HIDDEN)),
                  _full_spec((HIDDEN, HIDDEN))],
        out_specs=_rows_spec(HIDDEN),
        out_shape=jax.ShapeDtypeStruct((N_PAD, HIDDEN), jnp.float32),
    )(pdeg, pdeg, p1, p1, hs1, b1.reshape(1, HIDDEN), W2)

    p2 = _agg(hs2, src, dst)

    out = pl.pallas_call(
        _head_body,
        grid=(_GRID,),
        in_specs=[_half0_spec(), _half1_spec(), _half0_spec(), _half1_spec(),
                  _rows_spec(HIDDEN), _full_spec((1, HIDDEN)),
                  _full_spec((HIDDEN, 1)), _full_spec((1, 1))],
        out_specs=_rows_spec(1),
        out_shape=jax.ShapeDtypeStruct((N_PAD, 1), jnp.float32),
    )(pdeg, pdeg, p2, p2, hs2, b2.reshape(1, HIDDEN), Wlin, blin.reshape(1, 1))

    return out[:N_NODES, 0]


# speedup vs baseline: 21.8773x; 1.1101x over previous
import functools

import jax
import jax.numpy as jnp
from jax import lax
from jax.experimental import pallas as pl
from jax.experimental.pallas import tpu as pltpu
from jax.experimental.pallas import tpu_sc as plsc

N_NODES = 10000
D_FEAT = 128
HIDDEN = 128
N_EDGES = 320000
N_PAD = 10240

NC = 2
NS = 16
NW = NC * NS
EDGES_PER_W = N_EDGES // NW
CHUNK = 80
N_CHUNKS = EDGES_PER_W // CHUNK
ROWS_PER_TILE = N_PAD // NS

N_PAIRS = (N_CHUNKS - 1) // 2

_mesh = plsc.VectorSubcoreMesh(core_axis_name="c", subcore_axis_name="s")


DEGW = 128


@functools.partial(
    pl.kernel,
    out_type=jax.ShapeDtypeStruct((NC * N_PAD, DEGW), jnp.float32),
    mesh=_mesh,
    scratch_types=[
        pltpu.VMEM((CHUNK,), jnp.int32),
        pltpu.VMEM((CHUNK,), jnp.int32),
        pltpu.VMEM((CHUNK, DEGW), jnp.float32),
        pltpu.VMEM_SHARED((N_PAD, DEGW), jnp.float32),
        pltpu.SemaphoreType.DMA,
    ],
)
def _deg(ones_hbm, dst_hbm, out_hbm, dA, dB, rows_v, acc, sem_s):
    cid = lax.axis_index("c")
    sid = lax.axis_index("s")
    wid = sid * NC + cid
    r0 = sid * ROWS_PER_TILE
    o0 = cid * N_PAD
    e0 = wid * EDGES_PER_W

    pltpu.sync_copy(ones_hbm, rows_v)

    def init(k, _):
        pltpu.sync_copy(rows_v, acc.at[pl.ds(r0 + k * CHUNK, CHUNK)])
        return 0

    lax.fori_loop(0, ROWS_PER_TILE // CHUNK, init, 0)
    pltpu.sync_copy(dst_hbm.at[pl.ds(e0, CHUNK)], dA)
    plsc.subcore_barrier()

    def _wait():
        pltpu.make_async_copy(ones_hbm, rows_v, sem_s).wait()

    def body(k, _):
        j0 = 2 * k
        pltpu.async_copy(rows_v, acc.at[dA], sem_s, add=True)

        @pl.when(k > 0)
        def _wait_odd():
            _wait()

        pltpu.sync_copy(dst_hbm.at[pl.ds(e0 + (j0 + 1) * CHUNK, CHUNK)], dB)
        pltpu.async_copy(rows_v, acc.at[dB], sem_s, add=True)
        _wait()
        pltpu.sync_copy(dst_hbm.at[pl.ds(e0 + (j0 + 2) * CHUNK, CHUNK)], dA)
        return 0

    lax.fori_loop(0, N_PAIRS, body, 0)
    pltpu.async_copy(rows_v, acc.at[dA], sem_s, add=True)
    _wait()
    _wait()
    plsc.subcore_barrier()

    def writeback(k, _):
        b = r0 + k * CHUNK
        pltpu.sync_copy(acc.at[pl.ds(b, CHUNK)], rows_v)
        pltpu.sync_copy(rows_v, out_hbm.at[pl.ds(o0 + b, CHUNK)])
        return 0

    lax.fori_loop(0, ROWS_PER_TILE // CHUNK, writeback, 0)


@functools.partial(
    pl.kernel,
    out_type=jax.ShapeDtypeStruct((NC * N_PAD, HIDDEN), jnp.float32),
    mesh=_mesh,
    scratch_types=[
        pltpu.VMEM((CHUNK,), jnp.int32),
        pltpu.VMEM((CHUNK,), jnp.int32),
        pltpu.VMEM((CHUNK,), jnp.int32),
        pltpu.VMEM((CHUNK,), jnp.int32),
        pltpu.VMEM((CHUNK, HIDDEN), jnp.float32),
        pltpu.VMEM((CHUNK, HIDDEN), jnp.float32),
        pltpu.VMEM_SHARED((N_PAD, HIDDEN), jnp.float32),
        pltpu.SemaphoreType.DMA,
        pltpu.SemaphoreType.DMA,
    ],
)
def _agg(hs_hbm, src_hbm, dst_hbm, out_hbm, sA, sB, dA, dB, bufA, bufB, acc,
         sem_g, sem_s):
    cid = lax.axis_index("c")
    sid = lax.axis_index("s")
    wid = sid * NC + cid
    r0 = sid * ROWS_PER_TILE
    o0 = cid * N_PAD
    e0 = wid * EDGES_PER_W

    def init(k, _):
        b = r0 + k * CHUNK
        pltpu.sync_copy(hs_hbm.at[pl.ds(b, CHUNK)], bufA)
        pltpu.sync_copy(bufA, acc.at[pl.ds(b, CHUNK)])
        return 0

    lax.fori_loop(0, ROWS_PER_TILE // CHUNK, init, 0)

    pltpu.sync_copy(src_hbm.at[pl.ds(e0, CHUNK)], sA)
    pltpu.async_copy(hs_hbm.at[sA], bufA, sem_g)
    plsc.subcore_barrier()

    def _wait(sem):
        pltpu.make_async_copy(hs_hbm.at[pl.ds(0, CHUNK)], bufA, sem).wait()

    def body(k, _):
        j0 = 2 * k
        pltpu.sync_copy(dst_hbm.at[pl.ds(e0 + j0 * CHUNK, CHUNK)], dA)

        @pl.when(k > 0)
        def _wait_prev_scatter():
            _wait(sem_s)

        pltpu.sync_copy(src_hbm.at[pl.ds(e0 + (j0 + 1) * CHUNK, CHUNK)], sB)
        pltpu.async_copy(hs_hbm.at[sB], bufB, sem_g)
        _wait(sem_g)
        pltpu.async_copy(bufA, acc.at[dA], sem_s, add=True)
        pltpu.sync_copy(dst_hbm.at[pl.ds(e0 + (j0 + 1) * CHUNK, CHUNK)], dB)
        pltpu.sync_copy(src_hbm.at[pl.ds(e0 + (j0 + 2) * CHUNK, CHUNK)], sA)
        _wait(sem_s)
        pltpu.async_copy(hs_hbm.at[sA], bufA, sem_g)
        _wait(sem_g)
        pltpu.async_copy(bufB, acc.at[dB], sem_s, add=True)
        return 0

    lax.fori_loop(0, N_PAIRS, body, 0)

    pltpu.sync_copy(dst_hbm.at[pl.ds(e0 + (N_CHUNKS - 1) * CHUNK, CHUNK)], dA)
    _wait(sem_s)
    _wait(sem_g)
    pltpu.async_copy(bufA, acc.at[dA], sem_s, add=True)
    _wait(sem_s)
    plsc.subcore_barrier()

    def writeback(k, _):
        b = r0 + k * CHUNK
        pltpu.sync_copy(acc.at[pl.ds(b, CHUNK)], bufA)
        pltpu.sync_copy(bufA, out_hbm.at[pl.ds(o0 + b, CHUNK)])
        return 0

    lax.fori_loop(0, ROWS_PER_TILE // CHUNK, writeback, 0)


ROWS_BLK = 1024
_GRID = N_PAD // ROWS_BLK
_NBLK = N_PAD // ROWS_BLK


def _scale1_body(pd0_ref, pd1_ref, x_ref, w_ref, o_ref):
    dinv = lax.rsqrt(pd0_ref[...] + pd1_ref[...] - 1.0)[:, :1]
    h = jnp.dot(x_ref[...], w_ref[...], preferred_element_type=jnp.float32)
    o_ref[...] = h * dinv


def _mid_body(pd0_ref, pd1_ref, p0_ref, p1_ref, hs_ref, b_ref, w_ref, o_ref):
    dinv = lax.rsqrt(pd0_ref[...] + pd1_ref[...] - 1.0)[:, :1]
    agg = p0_ref[...] + p1_ref[...] - hs_ref[...]
    h = jnp.maximum(agg * dinv + b_ref[...], 0.0)
    o_ref[...] = jnp.dot(h, w_ref[...], preferred_element_type=jnp.float32) * dinv


def _head_body(pd0_ref, pd1_ref, p0_ref, p1_ref, hs_ref, b_ref, w_ref,
               blin_ref, o_ref):
    dinv = lax.rsqrt(pd0_ref[...] + pd1_ref[...] - 1.0)[:, :1]
    agg = p0_ref[...] + p1_ref[...] - hs_ref[...]
    h = jnp.maximum(agg * dinv + b_ref[...], 0.0)
    o_ref[...] = (jnp.dot(h, w_ref[...], preferred_element_type=jnp.float32)
                  + blin_ref[...])


def _half0_spec(width=HIDDEN):
    return pl.BlockSpec((ROWS_BLK, width), lambda i: (i, 0))


def _half1_spec(width=HIDDEN):
    return pl.BlockSpec((ROWS_BLK, width), lambda i: (i + _NBLK, 0))


def _rows_spec(width):
    return pl.BlockSpec((ROWS_BLK, width), lambda i: (i, 0))


def _full_spec(shape):
    return pl.BlockSpec(shape, lambda i: tuple(0 for _ in shape))


def kernel(x, edge_index, W1, b1, W2, b2, Wlin, blin):
    src = edge_index[0].astype(jnp.int32)
    dst = edge_index[1].astype(jnp.int32)
    xp = jnp.pad(x, ((0, N_PAD - N_NODES), (0, 0)))
    ones = jnp.ones((CHUNK, DEGW), jnp.float32)

    pdeg = _deg(ones, dst)

    hs1 = pl.pallas_call(
        _scale1_body,
        grid=(_GRID,),
        in_specs=[_half0_spec(DEGW), _half1_spec(DEGW), _rows_spec(D_FEAT),
                  _full_spec((D_FEAT, HIDDEN))],
        out_specs=_rows_spec(HIDDEN),
        out_shape=jax.ShapeDtypeStruct((N_PAD, HIDDEN), jnp.float32),
    )(pdeg, pdeg, xp, W1)

    p1 = _agg(hs1, src, dst)

    hs2 = pl.pallas_call(
        _mid_body,
        grid=(_GRID,),
        in_specs=[_half0_spec(DEGW), _half1_spec(DEGW), _half0_spec(), _half1_spec(),
                  _rows_spec(HIDDEN), _full_spec((1, HIDDEN)),
                  _full_spec((HIDDEN, HIDDEN))],
        out_specs=_rows_spec(HIDDEN),
        out_shape=jax.ShapeDtypeStruct((N_PAD, HIDDEN), jnp.float32),
    )(pdeg, pdeg, p1, p1, hs1, b1.reshape(1, HIDDEN), W2)

    p2 = _agg(hs2, src, dst)

    out = pl.pallas_call(
        _head_body,
        grid=(_GRID,),
        in_specs=[_half0_spec(DEGW), _half1_spec(DEGW), _half0_spec(), _half1_spec(),
                  _rows_spec(HIDDEN), _full_spec((1, HIDDEN)),
                  _full_spec((HIDDEN, 1)), _full_spec((1, 1))],
        out_specs=_rows_spec(1),
        out_shape=jax.ShapeDtypeStruct((N_PAD, 1), jnp.float32),
    )(pdeg, pdeg, p2, p2, hs2, b2.reshape(1, HIDDEN), Wlin, blin.reshape(1, 1))

    return out[:N_NODES, 0]
